# baseline (speedup 1.0000x reference)
import jax
import jax.numpy as jnp
from jax import lax
from jax.experimental import pallas as pl
from jax.experimental.pallas import tpu as pltpu
from jax.experimental.pallas import tpu_sc as plsc

NUM_ROWS = 160000
NUM_COLS = 256
N_SEG = 10000
N_SEG_PAD = 10240

NC = 2
NS = 16
L = 16

COLS_PER_SC = NUM_COLS // NC
ROWS_PER_TILE = NUM_ROWS // NS
CHUNK = 32
N_MAIN = ROWS_PER_TILE // CHUNK
TAIL = ROWS_PER_TILE - N_MAIN * CHUNK
NBUF = 8
N_GROUPS = N_MAIN // NBUF
LOAD_AHEAD = 4
SEG_PER_TILE = N_SEG_PAD // NS
STRIP = CHUNK
N_STRIPS = SEG_PER_TILE // STRIP
LAST_SEGS = N_SEG - (NS - 1) * SEG_PER_TILE
LAST_FULL_STRIPS = LAST_SEGS // STRIP
LAST_PART = LAST_SEGS - LAST_FULL_STRIPS * STRIP


def _body(x_hbm, idx_hbm, dx_hbm, out_hbm,
          idxb, rowsb, idxt_v, tail_v, ones_v, cnt_v, cntb1,
          acc_sh, cntacc_sh, sems, out_sems, rsems):
    c = lax.axis_index("c")
    s = lax.axis_index("s")
    seg0 = s * SEG_PER_TILE
    row_base = s * ROWS_PER_TILE
    col0 = c * COLS_PER_SC

    zeros16 = jnp.zeros((L,), jnp.float32)
    ones16 = jnp.ones((L,), jnp.float32)

    def ones_body(i, _):
        ones_v[i, :] = ones16
        return 0

    lax.fori_loop(0, CHUNK, ones_body, 0)

    rg_base = s * (ROWS_PER_TILE // 8)
    RG_PER_CHUNK = CHUNK // 8

    def start_load(k, b):
        rg0 = rg_base + k * RG_PER_CHUNK
        for i in range(RG_PER_CHUNK):
            pltpu.async_copy(x_hbm.at[rg0 + i, c],
                             rowsb[b].at[pl.ds(i * 8, 8)], sems[b])
        pltpu.async_copy(idx_hbm.at[pl.ds(row_base + k * CHUNK, CHUNK)],
                         idxb[b], sems[b])

    def wait_load(b):
        pltpu.make_async_copy(dx_hbm.at[pl.ds(0, CHUNK)], rowsb[b],
                              sems[b]).wait()
        pltpu.make_async_copy(idx_hbm.at[pl.ds(row_base, CHUNK)],
                              idxb[b], sems[b]).wait()

    def start_scatter(k, b):
        pltpu.async_copy(rowsb[b], acc_sh.at[idxb[b]], sems[NBUF + b],
                         add=True)
        pltpu.async_copy(ones_v, cntacc_sh.at[idxb[b]], sems[NBUF + b],
                         add=True)

    def wait_scatter(b):
        pltpu.make_async_copy(rowsb[b], acc_sh.at[idxb[b]],
                              sems[NBUF + b]).wait()
        pltpu.make_async_copy(ones_v, cntacc_sh.at[idxb[b]],
                              sems[NBUF + b]).wait()

    for b in range(LOAD_AHEAD):
        start_load(b, b)

    def zero_body(i, _):
        for j in range(COLS_PER_SC // L):
            rowsb[NBUF - 1][i, pl.ds(j * L, L)] = zeros16
        cnt_v[i, :] = zeros16
        return 0

    lax.fori_loop(0, CHUNK, zero_body, 0)
    for st in range(N_STRIPS):
        pltpu.async_copy(rowsb[NBUF - 1],
                         acc_sh.at[pl.ds(seg0 + st * STRIP, STRIP)],
                         out_sems[0])
        pltpu.async_copy(cnt_v, cntacc_sh.at[pl.ds(seg0 + st * STRIP, STRIP)],
                         out_sems[1])
    for st in range(N_STRIPS):
        pltpu.make_async_copy(dx_hbm.at[pl.ds(0, CHUNK)], rowsb[NBUF - 1],
                              out_sems[0]).wait()
        pltpu.make_async_copy(dx_hbm.at[pl.ds(0, CHUNK // 8)],
                              rowsb[NBUF - 1].at[pl.ds(0, CHUNK // 8)],
                              out_sems[1]).wait()

    plsc.subcore_barrier()

    def slot(j, b, drain, load):
        wait_load(b)
        if drain:
            wait_scatter((b + LOAD_AHEAD) % NBUF)
        if load:
            start_load(j + LOAD_AHEAD, (b + LOAD_AHEAD) % NBUF)
        start_scatter(j, b)

    for b in range(NBUF):
        slot(b, b, b >= NBUF - LOAD_AHEAD, True)

    def group_body(g, _):
        j0 = g * NBUF
        for b in range(NBUF):
            slot(j0 + b, b, True, True)
        return 0

    lax.fori_loop(1, N_GROUPS - 1, group_body, 0)

    j0 = (N_GROUPS - 1) * NBUF
    for b in range(NBUF):
        slot(j0 + b, b, True, b < NBUF - LOAD_AHEAD)
    for b in range(LOAD_AHEAD, NBUF):
        wait_scatter(b)

    pltpu.sync_copy(idx_hbm.at[pl.ds(row_base + N_MAIN * CHUNK, TAIL)],
                    idxt_v)
    rg_tail = rg_base + N_MAIN * RG_PER_CHUNK
    for i in range(TAIL // 8):
        pltpu.sync_copy(x_hbm.at[rg_tail + i, c],
                        tail_v.at[pl.ds(i * 8, 8)])
    pltpu.sync_copy(tail_v, acc_sh.at[idxt_v], add=True)
    pltpu.sync_copy(ones_v.at[pl.ds(0, TAIL)], cntacc_sh.at[idxt_v], add=True)

    plsc.subcore_barrier()

    seg_rg0 = s * (SEG_PER_TILE // 8)

    def store_strip(st, b, n=STRIP):
        rg = seg_rg0 + st * (STRIP // 8)
        for i in range(n // 8):
            pltpu.async_copy(rowsb[b].at[pl.ds(i * 8, 8)],
                             out_hbm.at[rg + i, c], out_sems[b])

    def wait_strip(b, n=STRIP):
        pltpu.make_async_copy(rowsb[b].at[pl.ds(0, n)],
                              dx_hbm.at[pl.ds(0, n)], out_sems[b]).wait()

    cntb = (cnt_v, cntb1)

    def start_read(st, b):
        pltpu.async_copy(acc_sh.at[pl.ds(seg0 + st * STRIP, STRIP)],
                         rowsb[b], rsems[b])
        pltpu.async_copy(cntacc_sh.at[pl.ds(seg0 + st * STRIP, STRIP)],
                         cntb[b], rsems[b])

    def wait_read(b):
        pltpu.make_async_copy(dx_hbm.at[pl.ds(0, CHUNK)], rowsb[b],
                              rsems[b]).wait()
        pltpu.make_async_copy(dx_hbm.at[pl.ds(0, CHUNK // 8)],
                              rowsb[b].at[pl.ds(0, CHUNK // 8)], rsems[b]).wait()

    def drain_store(st_done):
        b = st_done % 2
        if st_done < LAST_FULL_STRIPS:
            wait_strip(b)
        else:
            @pl.when(s < NS - 1)
            def _():
                wait_strip(b)

            if st_done == LAST_FULL_STRIPS:
                @pl.when(s == NS - 1)
                def _():
                    wait_strip(b, LAST_PART)

    start_read(0, 0)
    for st in range(N_STRIPS):
        b = st % 2
        wait_read(b)
        if st + 1 < N_STRIPS:
            if st >= 1:
                drain_store(st - 1)
            start_read(st + 1, 1 - b)

        def div_body(i, _):
            inv = ones16 / jnp.maximum(cntb[b][i, :], ones16)
            for j in range(COLS_PER_SC // L):
                rowsb[b][i, pl.ds(j * L, L)] = (
                    rowsb[b][i, pl.ds(j * L, L)] * inv)
            return 0

        lax.fori_loop(0, STRIP, div_body, 0)

        if st < LAST_FULL_STRIPS:
            store_strip(st, b)
        else:
            @pl.when(s < NS - 1)
            def _():
                store_strip(st, b)

            if st == LAST_FULL_STRIPS:
                @pl.when(s == NS - 1)
                def _():
                    store_strip(st, b, LAST_PART)

    drain_store(N_STRIPS - 2)
    drain_store(N_STRIPS - 1)


def _mean_pool(x, index, dx):
    run = pl.kernel(
        _body,
        out_type=jax.ShapeDtypeStruct((N_SEG // 8, NC, 8, COLS_PER_SC),
                                      jnp.float32),
        mesh=plsc.VectorSubcoreMesh(core_axis_name="c", subcore_axis_name="s"),
        compiler_params=pltpu.CompilerParams(use_tc_tiling_on_sc=False),
        scratch_types=[
            [pltpu.VMEM((CHUNK,), jnp.int32) for _ in range(NBUF)],
            [pltpu.VMEM((CHUNK, COLS_PER_SC), jnp.float32)
             for _ in range(NBUF)],
            pltpu.VMEM((TAIL,), jnp.int32),
            pltpu.VMEM((TAIL, COLS_PER_SC), jnp.float32),
            pltpu.VMEM((CHUNK, L), jnp.float32),
            pltpu.VMEM((STRIP, L), jnp.float32),
            pltpu.VMEM((STRIP, L), jnp.float32),
            pltpu.VMEM_SHARED((N_SEG_PAD, COLS_PER_SC), jnp.float32),
            pltpu.VMEM_SHARED((N_SEG_PAD, L), jnp.float32),
            [pltpu.SemaphoreType.DMA for _ in range(2 * NBUF)],
            [pltpu.SemaphoreType.DMA for _ in range(2)],
            [pltpu.SemaphoreType.DMA for _ in range(2)],
        ],
    )
    return run(x, index, dx)


@jax.jit
def kernel(x, index):
    x4 = x.reshape(NUM_ROWS // 8, 8, NC, COLS_PER_SC).transpose(0, 2, 1, 3)
    dx = jnp.zeros((CHUNK, COLS_PER_SC), jnp.float32)
    out4 = _mean_pool(x4, index.astype(jnp.int32), dx)
    return out4.transpose(0, 2, 1, 3).reshape(N_SEG, NUM_COLS)


# speedup vs baseline: 1.0938x; 1.0938x over previous
import jax
import jax.numpy as jnp
from jax import lax
from jax.experimental import pallas as pl
from jax.experimental.pallas import tpu as pltpu
from jax.experimental.pallas import tpu_sc as plsc

NUM_ROWS = 160000
NUM_COLS = 256
N_SEG = 10000
N_SEG_PAD = 10240

NC = 2
NS = 16
L = 16

COLS_PER_SC = NUM_COLS // NC
ROWS_PER_TILE = NUM_ROWS // NS
CHUNK = 32
N_MAIN = ROWS_PER_TILE // CHUNK
TAIL = ROWS_PER_TILE - N_MAIN * CHUNK
NBUF = 8
N_GROUPS = N_MAIN // NBUF
LOAD_AHEAD = 5
SEG_PER_TILE = N_SEG_PAD // NS
STRIP = CHUNK
N_STRIPS = SEG_PER_TILE // STRIP
LAST_SEGS = N_SEG - (NS - 1) * SEG_PER_TILE
LAST_FULL_STRIPS = LAST_SEGS // STRIP
LAST_PART = LAST_SEGS - LAST_FULL_STRIPS * STRIP


def _body(x_hbm, idx_hbm, dx_hbm, out_hbm,
          idxb, rowsb, idxt_v, tail_v, ones_v, cnt_v, cntb1,
          acc_sh, cntacc_sh, sems, out_sems, rsems):
    c = lax.axis_index("c")
    s = lax.axis_index("s")
    seg0 = s * SEG_PER_TILE
    row_base = s * ROWS_PER_TILE
    col0 = c * COLS_PER_SC

    zeros16 = jnp.zeros((L,), jnp.float32)
    ones16 = jnp.ones((L,), jnp.float32)

    def ones_body(i, _):
        ones_v[i, :] = ones16
        return 0

    lax.fori_loop(0, CHUNK, ones_body, 0)

    rg_base = s * (ROWS_PER_TILE // 8)
    RG_PER_CHUNK = CHUNK // 8

    def start_load(k, b):
        rg0 = rg_base + k * RG_PER_CHUNK
        for i in range(RG_PER_CHUNK):
            pltpu.async_copy(x_hbm.at[rg0 + i, c],
                             rowsb[b].at[pl.ds(i * 8, 8)], sems[b])
        pltpu.async_copy(idx_hbm.at[pl.ds(row_base + k * CHUNK, CHUNK)],
                         idxb[b], sems[b])

    def wait_load(b):
        pltpu.make_async_copy(dx_hbm.at[pl.ds(0, CHUNK)], rowsb[b],
                              sems[b]).wait()
        pltpu.make_async_copy(idx_hbm.at[pl.ds(row_base, CHUNK)],
                              idxb[b], sems[b]).wait()

    def start_scatter(k, b):
        pltpu.async_copy(rowsb[b], acc_sh.at[idxb[b]], sems[NBUF + b],
                         add=True)
        pltpu.async_copy(ones_v, cntacc_sh.at[idxb[b]], sems[NBUF + b],
                         add=True)

    def wait_scatter(b):
        pltpu.make_async_copy(rowsb[b], acc_sh.at[idxb[b]],
                              sems[NBUF + b]).wait()
        pltpu.make_async_copy(ones_v, cntacc_sh.at[idxb[b]],
                              sems[NBUF + b]).wait()

    for b in range(LOAD_AHEAD):
        start_load(b, b)

    def zero_body(i, _):
        for j in range(COLS_PER_SC // L):
            rowsb[NBUF - 1][i, pl.ds(j * L, L)] = zeros16
        cnt_v[i, :] = zeros16
        return 0

    lax.fori_loop(0, CHUNK, zero_body, 0)
    for st in range(N_STRIPS):
        pltpu.async_copy(rowsb[NBUF - 1],
                         acc_sh.at[pl.ds(seg0 + st * STRIP, STRIP)],
                         out_sems[0])
        pltpu.async_copy(cnt_v, cntacc_sh.at[pl.ds(seg0 + st * STRIP, STRIP)],
                         out_sems[1])
    for st in range(N_STRIPS):
        pltpu.make_async_copy(dx_hbm.at[pl.ds(0, CHUNK)], rowsb[NBUF - 1],
                              out_sems[0]).wait()
        pltpu.make_async_copy(dx_hbm.at[pl.ds(0, CHUNK // 8)],
                              rowsb[NBUF - 1].at[pl.ds(0, CHUNK // 8)],
                              out_sems[1]).wait()

    plsc.subcore_barrier()

    def slot(j, b, drain, load):
        wait_load(b)
        if drain:
            wait_scatter((b + LOAD_AHEAD) % NBUF)
        if load:
            start_load(j + LOAD_AHEAD, (b + LOAD_AHEAD) % NBUF)
        start_scatter(j, b)

    for b in range(NBUF):
        slot(b, b, b >= NBUF - LOAD_AHEAD, True)

    def group_body(g, _):
        j0 = g * NBUF
        for b in range(NBUF):
            slot(j0 + b, b, True, True)
        return 0

    lax.fori_loop(1, N_GROUPS - 1, group_body, 0)

    j0 = (N_GROUPS - 1) * NBUF
    for b in range(NBUF):
        slot(j0 + b, b, True, b < NBUF - LOAD_AHEAD)
    for b in range(LOAD_AHEAD, NBUF):
        wait_scatter(b)

    pltpu.sync_copy(idx_hbm.at[pl.ds(row_base + N_MAIN * CHUNK, TAIL)],
                    idxt_v)
    rg_tail = rg_base + N_MAIN * RG_PER_CHUNK
    for i in range(TAIL // 8):
        pltpu.sync_copy(x_hbm.at[rg_tail + i, c],
                        tail_v.at[pl.ds(i * 8, 8)])
    pltpu.sync_copy(tail_v, acc_sh.at[idxt_v], add=True)
    pltpu.sync_copy(ones_v.at[pl.ds(0, TAIL)], cntacc_sh.at[idxt_v], add=True)

    plsc.subcore_barrier()

    seg_rg0 = s * (SEG_PER_TILE // 8)

    def store_strip(st, b, n=STRIP):
        rg = seg_rg0 + st * (STRIP // 8)
        for i in range(n // 8):
            pltpu.async_copy(rowsb[b].at[pl.ds(i * 8, 8)],
                             out_hbm.at[rg + i, c], out_sems[b])

    def wait_strip(b, n=STRIP):
        pltpu.make_async_copy(rowsb[b].at[pl.ds(0, n)],
                              dx_hbm.at[pl.ds(0, n)], out_sems[b]).wait()

    cntb = (cnt_v, cntb1)

    def start_read(st, b):
        pltpu.async_copy(acc_sh.at[pl.ds(seg0 + st * STRIP, STRIP)],
                         rowsb[b], rsems[b])
        pltpu.async_copy(cntacc_sh.at[pl.ds(seg0 + st * STRIP, STRIP)],
                         cntb[b], rsems[b])

    def wait_read(b):
        pltpu.make_async_copy(dx_hbm.at[pl.ds(0, CHUNK)], rowsb[b],
                              rsems[b]).wait()
        pltpu.make_async_copy(dx_hbm.at[pl.ds(0, CHUNK // 8)],
                              rowsb[b].at[pl.ds(0, CHUNK // 8)], rsems[b]).wait()

    def drain_store(st_done):
        b = st_done % 2
        if st_done < LAST_FULL_STRIPS:
            wait_strip(b)
        else:
            @pl.when(s < NS - 1)
            def _():
                wait_strip(b)

            if st_done == LAST_FULL_STRIPS:
                @pl.when(s == NS - 1)
                def _():
                    wait_strip(b, LAST_PART)

    start_read(0, 0)
    for st in range(N_STRIPS):
        b = st % 2
        wait_read(b)
        if st + 1 < N_STRIPS:
            if st >= 1:
                drain_store(st - 1)
            start_read(st + 1, 1 - b)

        def div_body(i, _):
            inv = ones16 / jnp.maximum(cntb[b][i, :], ones16)
            for j in range(COLS_PER_SC // L):
                rowsb[b][i, pl.ds(j * L, L)] = (
                    rowsb[b][i, pl.ds(j * L, L)] * inv)
            return 0

        lax.fori_loop(0, STRIP, div_body, 0)

        if st < LAST_FULL_STRIPS:
            store_strip(st, b)
        else:
            @pl.when(s < NS - 1)
            def _():
                store_strip(st, b)

            if st == LAST_FULL_STRIPS:
                @pl.when(s == NS - 1)
                def _():
                    store_strip(st, b, LAST_PART)

    drain_store(N_STRIPS - 2)
    drain_store(N_STRIPS - 1)


def _mean_pool(x, index, dx):
    run = pl.kernel(
        _body,
        out_type=jax.ShapeDtypeStruct((N_SEG // 8, NC, 8, COLS_PER_SC),
                                      jnp.float32),
        mesh=plsc.VectorSubcoreMesh(core_axis_name="c", subcore_axis_name="s"),
        compiler_params=pltpu.CompilerParams(use_tc_tiling_on_sc=False),
        scratch_types=[
            [pltpu.VMEM((CHUNK,), jnp.int32) for _ in range(NBUF)],
            [pltpu.VMEM((CHUNK, COLS_PER_SC), jnp.float32)
             for _ in range(NBUF)],
            pltpu.VMEM((TAIL,), jnp.int32),
            pltpu.VMEM((TAIL, COLS_PER_SC), jnp.float32),
            pltpu.VMEM((CHUNK, L), jnp.float32),
            pltpu.VMEM((STRIP, L), jnp.float32),
            pltpu.VMEM((STRIP, L), jnp.float32),
            pltpu.VMEM_SHARED((N_SEG_PAD, COLS_PER_SC), jnp.float32),
            pltpu.VMEM_SHARED((N_SEG_PAD, L), jnp.float32),
            [pltpu.SemaphoreType.DMA for _ in range(2 * NBUF)],
            [pltpu.SemaphoreType.DMA for _ in range(2)],
            [pltpu.SemaphoreType.DMA for _ in range(2)],
        ],
    )
    return run(x, index, dx)


@jax.jit
def kernel(x, index):
    x4 = x.reshape(NUM_ROWS // 8, 8, NC, COLS_PER_SC).transpose(0, 2, 1, 3)
    dx = jnp.zeros((CHUNK, COLS_PER_SC), jnp.float32)
    out4 = _mean_pool(x4, index.astype(jnp.int32), dx)
    return out4.transpose(0, 2, 1, 3).reshape(N_SEG, NUM_COLS)
